# baseline (speedup 1.0000x reference)
import functools

import jax
import jax.numpy as jnp
from jax import lax
from jax.experimental import pallas as pl
from jax.experimental.pallas import tpu as pltpu
from jax.experimental.pallas import tpu_sc as plsc

_NC = 2
_NS = 16
_NW = _NC * _NS
_IDXW = 128
_CHR = 16
_BK = _CHR * _IDXW


def _norm_body(x_ref, sc_ref, bi_ref, root_ref, rb_ref, h_ref, r_ref):
    x = x_ref[...]
    mu = jnp.mean(x, axis=1, keepdims=True)
    var = jnp.mean((x - mu) ** 2, axis=1, keepdims=True)
    h = (x - mu) / jnp.sqrt(var + 1e-5) * sc_ref[...] + bi_ref[...]
    h = jnp.where(h > 0, h, 0.01 * h)
    h_ref[...] = h
    r_ref[...] = x + jnp.dot(h, root_ref[...],
                             preferred_element_type=jnp.float32) + rb_ref[...]


def _edge_body(ea_t_ref, hs_ref, w1t_ref, b1c_ref, w2t_ref, b2c_ref, out_ref):
    d2 = w2t_ref.shape[0]
    d = int(round(d2 ** 0.5))
    bk = ea_t_ref.shape[1]
    et = jnp.dot(w1t_ref[...], ea_t_ref[...],
                 preferred_element_type=jnp.float32)
    et = et + b1c_ref[...]
    et = jnp.where(et > 0, et, 0.01 * et)
    pt = jnp.dot(w2t_ref[...], et,
                 preferred_element_type=jnp.float32) + b2c_ref[...]
    hsbt = hs_ref[...].T
    sub8 = bk // 8
    msg_k = []
    for k in range(8):
        acc = None
        for dd in range(d):
            hrow = jnp.broadcast_to(hsbt[d * k + dd:d * k + dd + 1, :],
                                    (d, sub8))
            term = hrow * pt[dd * d:(dd + 1) * d, k * sub8:(k + 1) * sub8]
            acc = term if acc is None else acc + term
        msg_k.append(acc)
    msgt = jnp.concatenate(
        [jnp.concatenate(msg_k, axis=1),
         jnp.ones((1, bk), jnp.float32),
         jnp.zeros((d - 1, bk), jnp.float32)], axis=0)
    sub4 = bk // 4
    out_ref[...] = jnp.concatenate(
        [msgt[:, j * sub4:(j + 1) * sub4].T for j in range(4)],
        axis=1)


def _final_body(r_ref, *p_refs):
    out_ref = p_refs[-1]
    s = p_refs[0][...]
    for pr in p_refs[1:-1]:
        s = s + pr[...]
    d = r_ref.shape[1]
    aggr = s[:, :d] / jnp.maximum(s[:, d:d + 1], 1.0)
    out_ref[...] = r_ref[...] + aggr


def _make_gather(n_rows_per_w, n_chunks, chr_, row_off, ep_h, d, n_nodes,
                 fill_tiles):
    mesh = plsc.VectorSubcoreMesh(core_axis_name="c", subcore_axis_name="s")
    fpr = n_nodes // fill_tiles

    @functools.partial(
        pl.kernel, mesh=mesh,
        compiler_params=pltpu.CompilerParams(
            use_tc_tiling_on_sc=False, needs_layout_passes=False),
        out_type=jax.ShapeDtypeStruct((ep_h, d), jnp.float32),
        scratch_types=[
            pltpu.VMEM((chr_, _IDXW), jnp.int32),
            pltpu.VMEM((chr_ * _IDXW, d), jnp.float32),
            pltpu.VMEM((_BK,), jnp.int32),
            pltpu.VMEM_SHARED((n_nodes, d), jnp.float32),
            pltpu.SemaphoreType.DMA,
        ],
    )
    def gather_k(h_hbm, src_hbm, out_hbm, idx_v, rows_v, stage_v, h_sh, sem):
        sid = lax.axis_index("s")
        wid = sid * _NC + lax.axis_index("c")

        @pl.when(sid < fill_tiles)
        def _():
            pltpu.sync_copy(h_hbm.at[pl.ds(sid * fpr, fpr)],
                            h_sh.at[pl.ds(sid * fpr, fpr)])

        plsc.subcore_barrier()

        iota = lax.iota(jnp.int32, 16)
        evec = (iota & 7) * (_BK // 8) + (iota >> 3)
        rpb = _BK // _IDXW

        def chunk(c, carry):
            lrow = wid * n_rows_per_w + c * chr_
            grow = row_off + lrow
            g = grow // rpb
            jb = grow - g * rpb
            pltpu.sync_copy(src_hbm.at[0, pl.ds(g * _BK, _BK)], stage_v)
            for v in range(chr_ * 8):
                p0 = jb * _IDXW + v * 16
                vals = plsc.load_gather(stage_v, [evec + (p0 >> 3)])
                idx_v[v // 8, pl.ds((v % 8) * 16, 16)] = vals
            copies = [
                pltpu.async_copy(h_sh.at[idx_v.at[j]],
                                 rows_v.at[pl.ds(j * _IDXW, _IDXW)], sem)
                for j in range(chr_)
            ]
            for cp in copies:
                cp.wait()
            pltpu.sync_copy(rows_v,
                            out_hbm.at[pl.ds(lrow * _IDXW, chr_ * _IDXW)])
            return carry

        lax.fori_loop(0, n_chunks, chunk, 0)

    return gather_k


def _make_scatter(n_rows_per_w, n_chunks, chr_, row_off, np_pad, w):
    mesh = plsc.VectorSubcoreMesh(core_axis_name="c", subcore_axis_name="s")
    zr = np_pad // _NS

    @functools.partial(
        pl.kernel, mesh=mesh,
        compiler_params=pltpu.CompilerParams(
            use_tc_tiling_on_sc=False, needs_layout_passes=False),
        out_type=jax.ShapeDtypeStruct((_NC, np_pad, w), jnp.float32),
        scratch_types=[
            pltpu.VMEM((chr_, _IDXW), jnp.int32),
            pltpu.VMEM((chr_ * _IDXW, w), jnp.float32),
            pltpu.VMEM((_BK,), jnp.int32),
            pltpu.VMEM_SHARED((np_pad, w), jnp.float32),
            pltpu.SemaphoreType.DMA,
        ],
    )
    def scatter_k(msg_hbm, dst_hbm, zero_hbm, out_hbm, idx_v, buf_v, stage_v,
                  acc_sh, sem):
        cid = lax.axis_index("c")
        sid = lax.axis_index("s")
        wid = sid * _NC + cid
        pltpu.sync_copy(zero_hbm.at[pl.ds(sid * zr, zr)],
                        acc_sh.at[pl.ds(sid * zr, zr)])
        plsc.subcore_barrier()

        iota = lax.iota(jnp.int32, 16)
        evec = (iota & 3) * (_BK // 4) + (iota >> 2)
        rpb = _BK // _IDXW

        def chunk(c, carry):
            lrow = wid * n_rows_per_w + c * chr_
            grow = row_off + lrow
            g = grow // rpb
            jb = grow - g * rpb
            pltpu.sync_copy(dst_hbm.at[1, pl.ds(g * _BK, _BK)], stage_v)
            for v in range(chr_ * 8):
                p0 = jb * _IDXW + v * 16
                vals = plsc.load_gather(stage_v, [evec + (p0 >> 2)])
                idx_v[v // 8, pl.ds((v % 8) * 16, 16)] = vals
            pltpu.sync_copy(msg_hbm.at[pl.ds(lrow * _IDXW, chr_ * _IDXW)],
                            buf_v)
            for j in range(chr_):
                pltpu.sync_copy(buf_v.at[pl.ds(j * _IDXW, _IDXW)],
                                acc_sh.at[idx_v.at[j]], add=True)
            return carry

        lax.fori_loop(0, n_chunks, chunk, 0)
        plsc.subcore_barrier()
        pltpu.sync_copy(acc_sh.at[pl.ds(sid * zr, zr)],
                        out_hbm.at[cid, pl.ds(sid * zr, zr)])

    return scatter_k


def kernel(x, edge_index, edge_attr, ln_scale, ln_bias, w1, b1, w2, b2, root,
           bias):
    n, d = x.shape
    e_cnt, de = edge_attr.shape
    hdim = w1.shape[1]
    w = 2 * d

    grain = _BK * _NW
    ep = ((e_cnt + grain - 1) // grain) * grain
    rows = ep // _IDXW
    rpw = rows // _NW
    n_chunks = rpw // _CHR
    nb = ep // _BK
    np_pad = ((n + 1 + _NS * 8 - 1) // (_NS * 8)) * (_NS * 8)

    pad = ep - e_cnt
    ei_pad = jnp.concatenate(
        [edge_index,
         jnp.stack([jnp.zeros((pad,), jnp.int32),
                    jnp.full((pad,), n, jnp.int32)])], axis=1)
    ea_t = edge_attr.T

    zeros_acc = jnp.zeros((np_pad, w), jnp.float32)

    bn = 2000
    assert n % bn == 0
    h, r = pl.pallas_call(
        _norm_body,
        grid=(n // bn,),
        in_specs=[
            pl.BlockSpec((bn, d), lambda i: (i, 0)),
            pl.BlockSpec((1, d), lambda i: (0, 0)),
            pl.BlockSpec((1, d), lambda i: (0, 0)),
            pl.BlockSpec((d, d), lambda i: (0, 0)),
            pl.BlockSpec((1, d), lambda i: (0, 0)),
        ],
        out_specs=[
            pl.BlockSpec((bn, d), lambda i: (i, 0)),
            pl.BlockSpec((bn, d), lambda i: (i, 0)),
        ],
        out_shape=[
            jax.ShapeDtypeStruct((n, d), jnp.float32),
            jax.ShapeDtypeStruct((n, d), jnp.float32),
        ],
    )(x, ln_scale.reshape(1, d), ln_bias.reshape(1, d), root,
      bias.reshape(1, d))

    fill_tiles = 10
    assert n % fill_tiles == 0 and (n // fill_tiles) % 8 == 0
    nh = 4
    ep_h = ep // nh
    rows_h = rows // nh
    rpw_h = rows_h // _NW
    chr_h = 4
    assert rpw_h % chr_h == 0
    n_chunks_h = rpw_h // chr_h
    nbh = ep_h // _BK
    parts = []
    for half in range(nh):
        hs_h = _make_gather(rpw_h, n_chunks_h, chr_h, half * rows_h, ep_h, d,
                            n, fill_tiles)(h, ei_pad)
        hs128 = hs_h.reshape(ep_h * d // 128, 128)
        live = min(ep_h, max(0, e_cnt - half * ep_h))
        nb_live = (live + _BK - 1) // _BK
        msg128 = pl.pallas_call(
            _edge_body,
            grid=(nb_live,),
            in_specs=[
                pl.BlockSpec((de, _BK),
                             lambda i, o=half * nbh: (0, i + o)),
                pl.BlockSpec((_BK * d // 128, 128), lambda i: (i, 0)),
                pl.BlockSpec((hdim, de), lambda i: (0, 0)),
                pl.BlockSpec((hdim, 1), lambda i: (0, 0)),
                pl.BlockSpec((d * d, hdim), lambda i: (0, 0)),
                pl.BlockSpec((d * d, 1), lambda i: (0, 0)),
            ],
            out_specs=pl.BlockSpec((_BK * w // 128, 128), lambda i: (i, 0)),
            out_shape=jax.ShapeDtypeStruct((ep_h * w // 128, 128),
                                           jnp.float32),
        )(ea_t, hs128, w1.T, b1.reshape(hdim, 1), w2.T, b2.reshape(d * d, 1))
        msg = msg128.reshape(ep_h, w)
        parts.append(
            _make_scatter(rpw_h, n_chunks_h, chr_h, half * rows_h, np_pad,
                          w)(msg, ei_pad, zeros_acc))

    out = pl.pallas_call(
        _final_body,
        grid=(n // bn,),
        in_specs=[pl.BlockSpec((bn, d), lambda i: (i, 0))] + [
            pl.BlockSpec((bn, w), lambda i: (i, 0))
            for _ in range(2 * nh)
        ],
        out_specs=pl.BlockSpec((bn, d), lambda i: (i, 0)),
        out_shape=jax.ShapeDtypeStruct((n, d), jnp.float32),
    )(r, *[p[c, :n] for p in parts for c in range(2)])
    return out


# speedup vs baseline: 1.0393x; 1.0393x over previous
import functools

import jax
import jax.numpy as jnp
from jax import lax
from jax.experimental import pallas as pl
from jax.experimental.pallas import tpu as pltpu
from jax.experimental.pallas import tpu_sc as plsc

_NC = 2
_NS = 16
_NW = _NC * _NS
_IDXW = 128
_CHR = 16
_BK = _CHR * _IDXW


def _norm_body(x_ref, sc_ref, bi_ref, root_ref, rb_ref, h_ref, r_ref):
    x = x_ref[...]
    mu = jnp.mean(x, axis=1, keepdims=True)
    var = jnp.mean((x - mu) ** 2, axis=1, keepdims=True)
    h = (x - mu) / jnp.sqrt(var + 1e-5) * sc_ref[...] + bi_ref[...]
    h = jnp.where(h > 0, h, 0.01 * h)
    h_ref[...] = h
    r_ref[...] = x + jnp.dot(h, root_ref[...],
                             preferred_element_type=jnp.float32) + rb_ref[...]


def _edge_body(ea_t_ref, hs_ref, w1t_ref, b1c_ref, w2t_ref, b2c_ref, out_ref):
    d2 = w2t_ref.shape[0]
    d = int(round(d2 ** 0.5))
    bk = ea_t_ref.shape[1]
    et = jnp.dot(w1t_ref[...], ea_t_ref[...],
                 preferred_element_type=jnp.float32)
    et = et + b1c_ref[...]
    et = jnp.where(et > 0, et, 0.01 * et)
    pt = jnp.dot(w2t_ref[...], et,
                 preferred_element_type=jnp.float32) + b2c_ref[...]
    hsbt = hs_ref[...].T
    sub8 = bk // 8
    msg_k = []
    for k in range(8):
        acc = None
        for dd in range(d):
            hrow = jnp.broadcast_to(hsbt[d * k + dd:d * k + dd + 1, :],
                                    (d, sub8))
            term = hrow * pt[dd * d:(dd + 1) * d, k * sub8:(k + 1) * sub8]
            acc = term if acc is None else acc + term
        msg_k.append(acc)
    msgt = jnp.concatenate(
        [jnp.concatenate(msg_k, axis=1),
         jnp.ones((1, bk), jnp.float32),
         jnp.zeros((d - 1, bk), jnp.float32)], axis=0)
    sub4 = bk // 4
    out_ref[...] = jnp.concatenate(
        [msgt[:, j * sub4:(j + 1) * sub4].T for j in range(4)],
        axis=1)


def _final_body(r_ref, *p_refs):
    out_ref = p_refs[-1]
    s = p_refs[0][...]
    for pr in p_refs[1:-1]:
        s = s + pr[...]
    d = r_ref.shape[1]
    aggr = s[:, :d] / jnp.maximum(s[:, d:d + 1], 1.0)
    out_ref[...] = r_ref[...] + aggr


def _make_gather(n_rows_per_w, n_chunks, chr_, row_off, ep_h, d, n_nodes,
                 fill_tiles):
    mesh = plsc.VectorSubcoreMesh(core_axis_name="c", subcore_axis_name="s")
    fpr = n_nodes // fill_tiles

    @functools.partial(
        pl.kernel, mesh=mesh,
        compiler_params=pltpu.CompilerParams(
            use_tc_tiling_on_sc=False, needs_layout_passes=False),
        out_type=jax.ShapeDtypeStruct((ep_h, d), jnp.float32),
        scratch_types=[
            pltpu.VMEM((chr_, _IDXW), jnp.int32),
            pltpu.VMEM((chr_ * _IDXW, d), jnp.float32),
            pltpu.VMEM((_BK,), jnp.int32),
            pltpu.VMEM_SHARED((n_nodes, d), jnp.float32),
            pltpu.SemaphoreType.DMA,
        ],
    )
    def gather_k(h_hbm, src_hbm, out_hbm, idx_v, rows_v, stage_v, h_sh, sem):
        sid = lax.axis_index("s")
        wid = sid * _NC + lax.axis_index("c")

        @pl.when(sid < fill_tiles)
        def _():
            pltpu.sync_copy(h_hbm.at[pl.ds(sid * fpr, fpr)],
                            h_sh.at[pl.ds(sid * fpr, fpr)])

        plsc.subcore_barrier()

        iota = lax.iota(jnp.int32, 16)
        evec = (iota & 7) * (_BK // 8) + (iota >> 3)
        rpb = _BK // _IDXW

        def chunk(c, carry):
            lrow = wid * n_rows_per_w + c * chr_
            grow = row_off + lrow
            g = grow // rpb
            jb = grow - g * rpb
            pltpu.sync_copy(src_hbm.at[0, pl.ds(g * _BK, _BK)], stage_v)
            for v in range(chr_ * 8):
                p0 = jb * _IDXW + v * 16
                vals = plsc.load_gather(stage_v, [evec + (p0 >> 3)])
                idx_v[v // 8, pl.ds((v % 8) * 16, 16)] = vals
            copies = [
                pltpu.async_copy(h_sh.at[idx_v.at[j]],
                                 rows_v.at[pl.ds(j * _IDXW, _IDXW)], sem)
                for j in range(chr_)
            ]
            for cp in copies:
                cp.wait()
            pltpu.sync_copy(rows_v,
                            out_hbm.at[pl.ds(lrow * _IDXW, chr_ * _IDXW)])
            return carry

        lax.fori_loop(0, n_chunks, chunk, 0)

    return gather_k


def _make_scatter(n_rows_per_w, n_chunks, chr_, row_off, np_pad, w):
    mesh = plsc.VectorSubcoreMesh(core_axis_name="c", subcore_axis_name="s")
    zr = np_pad // _NS

    @functools.partial(
        pl.kernel, mesh=mesh,
        compiler_params=pltpu.CompilerParams(
            use_tc_tiling_on_sc=False, needs_layout_passes=False),
        out_type=jax.ShapeDtypeStruct((_NC, np_pad, w), jnp.float32),
        scratch_types=[
            pltpu.VMEM((chr_, _IDXW), jnp.int32),
            pltpu.VMEM((chr_ * _IDXW, w), jnp.float32),
            pltpu.VMEM((_BK,), jnp.int32),
            pltpu.VMEM_SHARED((np_pad, w), jnp.float32),
            pltpu.SemaphoreType.DMA,
        ],
    )
    def scatter_k(msg_hbm, dst_hbm, zero_hbm, out_hbm, idx_v, buf_v, stage_v,
                  acc_sh, sem):
        cid = lax.axis_index("c")
        sid = lax.axis_index("s")
        wid = sid * _NC + cid
        pltpu.sync_copy(zero_hbm.at[pl.ds(sid * zr, zr)],
                        acc_sh.at[pl.ds(sid * zr, zr)])
        plsc.subcore_barrier()

        iota = lax.iota(jnp.int32, 16)
        evec = (iota & 3) * (_BK // 4) + (iota >> 2)
        rpb = _BK // _IDXW

        def chunk(c, carry):
            lrow = wid * n_rows_per_w + c * chr_
            grow = row_off + lrow
            g = grow // rpb
            jb = grow - g * rpb
            pltpu.sync_copy(dst_hbm.at[1, pl.ds(g * _BK, _BK)], stage_v)
            for v in range(chr_ * 8):
                p0 = jb * _IDXW + v * 16
                vals = plsc.load_gather(stage_v, [evec + (p0 >> 2)])
                idx_v[v // 8, pl.ds((v % 8) * 16, 16)] = vals
            pltpu.sync_copy(msg_hbm.at[pl.ds(lrow * _IDXW, chr_ * _IDXW)],
                            buf_v)
            for j in range(chr_):
                pltpu.sync_copy(buf_v.at[pl.ds(j * _IDXW, _IDXW)],
                                acc_sh.at[idx_v.at[j]], add=True)
            return carry

        lax.fori_loop(0, n_chunks, chunk, 0)
        plsc.subcore_barrier()
        pltpu.sync_copy(acc_sh.at[pl.ds(sid * zr, zr)],
                        out_hbm.at[cid, pl.ds(sid * zr, zr)])

    return scatter_k


def kernel(x, edge_index, edge_attr, ln_scale, ln_bias, w1, b1, w2, b2, root,
           bias):
    n, d = x.shape
    e_cnt, de = edge_attr.shape
    hdim = w1.shape[1]
    w = 2 * d

    grain = _BK * _NW
    ep = ((e_cnt + grain - 1) // grain) * grain
    rows = ep // _IDXW
    rpw = rows // _NW
    n_chunks = rpw // _CHR
    nb = ep // _BK
    np_pad = ((n + 1 + _NS * 8 - 1) // (_NS * 8)) * (_NS * 8)

    pad = ep - e_cnt
    ei_pad = jnp.concatenate(
        [edge_index,
         jnp.stack([jnp.zeros((pad,), jnp.int32),
                    jnp.full((pad,), n, jnp.int32)])], axis=1)
    ea_t = edge_attr.T

    zeros_acc = jnp.zeros((np_pad, w), jnp.float32)

    bn = 2000
    assert n % bn == 0
    h, r = pl.pallas_call(
        _norm_body,
        grid=(n // bn,),
        in_specs=[
            pl.BlockSpec((bn, d), lambda i: (i, 0)),
            pl.BlockSpec((1, d), lambda i: (0, 0)),
            pl.BlockSpec((1, d), lambda i: (0, 0)),
            pl.BlockSpec((d, d), lambda i: (0, 0)),
            pl.BlockSpec((1, d), lambda i: (0, 0)),
        ],
        out_specs=[
            pl.BlockSpec((bn, d), lambda i: (i, 0)),
            pl.BlockSpec((bn, d), lambda i: (i, 0)),
        ],
        out_shape=[
            jax.ShapeDtypeStruct((n, d), jnp.float32),
            jax.ShapeDtypeStruct((n, d), jnp.float32),
        ],
    )(x, ln_scale.reshape(1, d), ln_bias.reshape(1, d), root,
      bias.reshape(1, d))

    fill_tiles = 10
    assert n % fill_tiles == 0 and (n // fill_tiles) % 8 == 0
    nh = 2
    ep_h = ep // nh
    rows_h = rows // nh
    rpw_h = rows_h // _NW
    chr_h = 8
    assert rpw_h % chr_h == 0
    n_chunks_h = rpw_h // chr_h
    nbh = ep_h // _BK
    parts = []
    for half in range(nh):
        hs_h = _make_gather(rpw_h, n_chunks_h, chr_h, half * rows_h, ep_h, d,
                            n, fill_tiles)(h, ei_pad)
        hs128 = hs_h.reshape(ep_h * d // 128, 128)
        live = min(ep_h, max(0, e_cnt - half * ep_h))
        nb_live = (live + _BK - 1) // _BK
        msg128 = pl.pallas_call(
            _edge_body,
            grid=(nb_live,),
            in_specs=[
                pl.BlockSpec((de, _BK),
                             lambda i, o=half * nbh: (0, i + o)),
                pl.BlockSpec((_BK * d // 128, 128), lambda i: (i, 0)),
                pl.BlockSpec((hdim, de), lambda i: (0, 0)),
                pl.BlockSpec((hdim, 1), lambda i: (0, 0)),
                pl.BlockSpec((d * d, hdim), lambda i: (0, 0)),
                pl.BlockSpec((d * d, 1), lambda i: (0, 0)),
            ],
            out_specs=pl.BlockSpec((_BK * w // 128, 128), lambda i: (i, 0)),
            out_shape=jax.ShapeDtypeStruct((ep_h * w // 128, 128),
                                           jnp.float32),
        )(ea_t, hs128, w1.T, b1.reshape(hdim, 1), w2.T, b2.reshape(d * d, 1))
        msg = msg128.reshape(ep_h, w)
        parts.append(
            _make_scatter(rpw_h, n_chunks_h, chr_h, half * rows_h, np_pad,
                          w)(msg, ei_pad, zeros_acc))

    out = pl.pallas_call(
        _final_body,
        grid=(n // bn,),
        in_specs=[pl.BlockSpec((bn, d), lambda i: (i, 0))] + [
            pl.BlockSpec((bn, w), lambda i: (i, 0))
            for _ in range(2 * nh)
        ],
        out_specs=pl.BlockSpec((bn, d), lambda i: (i, 0)),
        out_shape=jax.ShapeDtypeStruct((n, d), jnp.float32),
    )(r, *[p[c, :n] for p in parts for c in range(2)])
    return out


# speedup vs baseline: 1.0792x; 1.0384x over previous
import functools

import jax
import jax.numpy as jnp
from jax import lax
from jax.experimental import pallas as pl
from jax.experimental.pallas import tpu as pltpu
from jax.experimental.pallas import tpu_sc as plsc

_NC = 2
_NS = 16
_NW = _NC * _NS
_IDXW = 128
_CHR = 16
_BK = _CHR * _IDXW


def _norm_body(x_ref, sc_ref, bi_ref, root_ref, rb_ref, h_ref, r_ref):
    x = x_ref[...]
    mu = jnp.mean(x, axis=1, keepdims=True)
    var = jnp.mean((x - mu) ** 2, axis=1, keepdims=True)
    h = (x - mu) / jnp.sqrt(var + 1e-5) * sc_ref[...] + bi_ref[...]
    h = jnp.where(h > 0, h, 0.01 * h)
    h_ref[...] = h
    r_ref[...] = x + jnp.dot(h, root_ref[...],
                             preferred_element_type=jnp.float32) + rb_ref[...]


def _edge_body(ea_t_ref, hs_ref, w1t_ref, b1c_ref, w2t_ref, b2c_ref, out_ref):
    d2 = w2t_ref.shape[0]
    d = int(round(d2 ** 0.5))
    bk = ea_t_ref.shape[1]
    et = jnp.dot(w1t_ref[...], ea_t_ref[...],
                 preferred_element_type=jnp.float32)
    et = et + b1c_ref[...]
    et = jnp.where(et > 0, et, 0.01 * et)
    pt = jnp.dot(w2t_ref[...], et,
                 preferred_element_type=jnp.float32) + b2c_ref[...]
    hsbt = hs_ref[...].T
    sub8 = bk // 8
    msg_k = []
    for k in range(8):
        acc = None
        for dd in range(d):
            hrow = jnp.broadcast_to(hsbt[d * k + dd:d * k + dd + 1, :],
                                    (d, sub8))
            term = hrow * pt[dd * d:(dd + 1) * d, k * sub8:(k + 1) * sub8]
            acc = term if acc is None else acc + term
        msg_k.append(acc)
    msgt = jnp.concatenate(msg_k, axis=1)
    sub4 = bk // 4
    rr = lax.broadcasted_iota(jnp.int32, (d, 2 * d), 0)
    cc = lax.broadcasted_iota(jnp.int32, (d, 2 * d), 1)
    sel = (rr == cc).astype(jnp.float32)
    crow = (lax.broadcasted_iota(jnp.int32, (1, 2 * d), 1) == d
            ).astype(jnp.float32)
    pieces = [
        lax.dot_general(msgt[:, j * sub4:(j + 1) * sub4], sel,
                        (((0,), (0,)), ((), ())),
                        preferred_element_type=jnp.float32) + crow
        for j in range(4)
    ]
    out_ref[...] = jnp.concatenate(pieces, axis=1)


def _final_body(r_ref, *p_refs):
    out_ref = p_refs[-1]
    s = p_refs[0][...]
    for pr in p_refs[1:-1]:
        s = s + pr[...]
    d = r_ref.shape[1]
    aggr = s[:, :d] / jnp.maximum(s[:, d:d + 1], 1.0)
    out_ref[...] = r_ref[...] + aggr


def _make_gather(n_rows_per_w, n_chunks, chr_, row_off, ep_h, d, n_nodes,
                 fill_tiles):
    mesh = plsc.VectorSubcoreMesh(core_axis_name="c", subcore_axis_name="s")
    fpr = n_nodes // fill_tiles

    @functools.partial(
        pl.kernel, mesh=mesh,
        compiler_params=pltpu.CompilerParams(
            use_tc_tiling_on_sc=False, needs_layout_passes=False),
        out_type=jax.ShapeDtypeStruct((ep_h, d), jnp.float32),
        scratch_types=[
            pltpu.VMEM((chr_, _IDXW), jnp.int32),
            pltpu.VMEM((chr_ * _IDXW, d), jnp.float32),
            pltpu.VMEM((_BK,), jnp.int32),
            pltpu.VMEM_SHARED((n_nodes, d), jnp.float32),
            pltpu.SemaphoreType.DMA,
        ],
    )
    def gather_k(h_hbm, src_hbm, out_hbm, idx_v, rows_v, stage_v, h_sh, sem):
        sid = lax.axis_index("s")
        wid = sid * _NC + lax.axis_index("c")

        @pl.when(sid < fill_tiles)
        def _():
            pltpu.sync_copy(h_hbm.at[pl.ds(sid * fpr, fpr)],
                            h_sh.at[pl.ds(sid * fpr, fpr)])

        plsc.subcore_barrier()

        iota = lax.iota(jnp.int32, 16)
        evec = (iota & 7) * (_BK // 8) + (iota >> 3)
        rpb = _BK // _IDXW

        def chunk(c, carry):
            lrow = wid * n_rows_per_w + c * chr_
            grow = row_off + lrow
            g = grow // rpb
            jb = grow - g * rpb
            pltpu.sync_copy(src_hbm.at[0, pl.ds(g * _BK, _BK)], stage_v)
            for v in range(chr_ * 8):
                p0 = jb * _IDXW + v * 16
                vals = plsc.load_gather(stage_v, [evec + (p0 >> 3)])
                idx_v[v // 8, pl.ds((v % 8) * 16, 16)] = vals
            copies = [
                pltpu.async_copy(h_sh.at[idx_v.at[j]],
                                 rows_v.at[pl.ds(j * _IDXW, _IDXW)], sem)
                for j in range(chr_)
            ]
            for cp in copies:
                cp.wait()
            pltpu.sync_copy(rows_v,
                            out_hbm.at[pl.ds(lrow * _IDXW, chr_ * _IDXW)])
            return carry

        lax.fori_loop(0, n_chunks, chunk, 0)

    return gather_k


def _make_scatter(n_rows_per_w, n_chunks, chr_, row_off, np_pad, w):
    mesh = plsc.VectorSubcoreMesh(core_axis_name="c", subcore_axis_name="s")
    zr = np_pad // _NS

    @functools.partial(
        pl.kernel, mesh=mesh,
        compiler_params=pltpu.CompilerParams(
            use_tc_tiling_on_sc=False, needs_layout_passes=False),
        out_type=jax.ShapeDtypeStruct((_NC, np_pad, w), jnp.float32),
        scratch_types=[
            pltpu.VMEM((chr_, _IDXW), jnp.int32),
            pltpu.VMEM((chr_ * _IDXW, w), jnp.float32),
            pltpu.VMEM((_BK,), jnp.int32),
            pltpu.VMEM_SHARED((np_pad, w), jnp.float32),
            pltpu.SemaphoreType.DMA,
        ],
    )
    def scatter_k(msg_hbm, dst_hbm, zero_hbm, out_hbm, idx_v, buf_v, stage_v,
                  acc_sh, sem):
        cid = lax.axis_index("c")
        sid = lax.axis_index("s")
        wid = sid * _NC + cid
        pltpu.sync_copy(zero_hbm.at[pl.ds(sid * zr, zr)],
                        acc_sh.at[pl.ds(sid * zr, zr)])
        plsc.subcore_barrier()

        iota = lax.iota(jnp.int32, 16)
        evec = (iota & 3) * (_BK // 4) + (iota >> 2)
        rpb = _BK // _IDXW

        def chunk(c, carry):
            lrow = wid * n_rows_per_w + c * chr_
            grow = row_off + lrow
            g = grow // rpb
            jb = grow - g * rpb
            pltpu.sync_copy(dst_hbm.at[1, pl.ds(g * _BK, _BK)], stage_v)
            for v in range(chr_ * 8):
                p0 = jb * _IDXW + v * 16
                vals = plsc.load_gather(stage_v, [evec + (p0 >> 2)])
                idx_v[v // 8, pl.ds((v % 8) * 16, 16)] = vals
            pltpu.sync_copy(msg_hbm.at[pl.ds(lrow * _IDXW, chr_ * _IDXW)],
                            buf_v)
            for j in range(chr_):
                pltpu.sync_copy(buf_v.at[pl.ds(j * _IDXW, _IDXW)],
                                acc_sh.at[idx_v.at[j]], add=True)
            return carry

        lax.fori_loop(0, n_chunks, chunk, 0)
        plsc.subcore_barrier()
        pltpu.sync_copy(acc_sh.at[pl.ds(sid * zr, zr)],
                        out_hbm.at[cid, pl.ds(sid * zr, zr)])

    return scatter_k


def kernel(x, edge_index, edge_attr, ln_scale, ln_bias, w1, b1, w2, b2, root,
           bias):
    n, d = x.shape
    e_cnt, de = edge_attr.shape
    hdim = w1.shape[1]
    w = 2 * d

    grain = _BK * _NW
    ep = ((e_cnt + grain - 1) // grain) * grain
    rows = ep // _IDXW
    rpw = rows // _NW
    n_chunks = rpw // _CHR
    nb = ep // _BK
    np_pad = ((n + 1 + _NS * 8 - 1) // (_NS * 8)) * (_NS * 8)

    pad = ep - e_cnt
    ei_pad = jnp.concatenate(
        [edge_index,
         jnp.stack([jnp.zeros((pad,), jnp.int32),
                    jnp.full((pad,), n, jnp.int32)])], axis=1)
    ea_t = edge_attr.T

    zeros_acc = jnp.zeros((np_pad, w), jnp.float32)

    bn = 2000
    assert n % bn == 0
    h, r = pl.pallas_call(
        _norm_body,
        grid=(n // bn,),
        in_specs=[
            pl.BlockSpec((bn, d), lambda i: (i, 0)),
            pl.BlockSpec((1, d), lambda i: (0, 0)),
            pl.BlockSpec((1, d), lambda i: (0, 0)),
            pl.BlockSpec((d, d), lambda i: (0, 0)),
            pl.BlockSpec((1, d), lambda i: (0, 0)),
        ],
        out_specs=[
            pl.BlockSpec((bn, d), lambda i: (i, 0)),
            pl.BlockSpec((bn, d), lambda i: (i, 0)),
        ],
        out_shape=[
            jax.ShapeDtypeStruct((n, d), jnp.float32),
            jax.ShapeDtypeStruct((n, d), jnp.float32),
        ],
    )(x, ln_scale.reshape(1, d), ln_bias.reshape(1, d), root,
      bias.reshape(1, d))

    fill_tiles = 10
    assert n % fill_tiles == 0 and (n // fill_tiles) % 8 == 0
    nh = 2
    ep_h = ep // nh
    rows_h = rows // nh
    rpw_h = rows_h // _NW
    chr_h = 8
    assert rpw_h % chr_h == 0
    n_chunks_h = rpw_h // chr_h
    nbh = ep_h // _BK
    parts = []
    for half in range(nh):
        hs_h = _make_gather(rpw_h, n_chunks_h, chr_h, half * rows_h, ep_h, d,
                            n, fill_tiles)(h, ei_pad)
        hs128 = hs_h.reshape(ep_h * d // 128, 128)
        live = min(ep_h, max(0, e_cnt - half * ep_h))
        nb_live = (live + _BK - 1) // _BK
        msg128 = pl.pallas_call(
            _edge_body,
            grid=(nb_live,),
            in_specs=[
                pl.BlockSpec((de, _BK),
                             lambda i, o=half * nbh: (0, i + o)),
                pl.BlockSpec((_BK * d // 128, 128), lambda i: (i, 0)),
                pl.BlockSpec((hdim, de), lambda i: (0, 0)),
                pl.BlockSpec((hdim, 1), lambda i: (0, 0)),
                pl.BlockSpec((d * d, hdim), lambda i: (0, 0)),
                pl.BlockSpec((d * d, 1), lambda i: (0, 0)),
            ],
            out_specs=pl.BlockSpec((_BK * w // 128, 128), lambda i: (i, 0)),
            out_shape=jax.ShapeDtypeStruct((ep_h * w // 128, 128),
                                           jnp.float32),
        )(ea_t, hs128, w1.T, b1.reshape(hdim, 1), w2.T, b2.reshape(d * d, 1))
        msg = msg128.reshape(ep_h, w)
        parts.append(
            _make_scatter(rpw_h, n_chunks_h, chr_h, half * rows_h, np_pad,
                          w)(msg, ei_pad, zeros_acc))

    out = pl.pallas_call(
        _final_body,
        grid=(n // bn,),
        in_specs=[pl.BlockSpec((bn, d), lambda i: (i, 0))] + [
            pl.BlockSpec((bn, w), lambda i: (i, 0))
            for _ in range(2 * nh)
        ],
        out_specs=pl.BlockSpec((bn, d), lambda i: (i, 0)),
        out_shape=jax.ShapeDtypeStruct((n, d), jnp.float32),
    )(r, *[p[c, :n] for p in parts for c in range(2)])
    return out


# speedup vs baseline: 1.0874x; 1.0076x over previous
import functools

import jax
import jax.numpy as jnp
from jax import lax
from jax.experimental import pallas as pl
from jax.experimental.pallas import tpu as pltpu
from jax.experimental.pallas import tpu_sc as plsc

_NC = 2
_NS = 16
_NW = _NC * _NS
_IDXW = 128
_CHR = 16
_BK = _CHR * _IDXW


def _norm_body(x_ref, sc_ref, bi_ref, root_ref, rb_ref, h_ref, r_ref):
    x = x_ref[...]
    mu = jnp.mean(x, axis=1, keepdims=True)
    var = jnp.mean((x - mu) ** 2, axis=1, keepdims=True)
    h = (x - mu) / jnp.sqrt(var + 1e-5) * sc_ref[...] + bi_ref[...]
    h = jnp.where(h > 0, h, 0.01 * h)
    h_ref[...] = h
    r_ref[...] = x + jnp.dot(h, root_ref[...],
                             preferred_element_type=jnp.float32) + rb_ref[...]


def _edge_body(ea_t_ref, hs_ref, w1t_ref, b1c_ref, w2t_ref, b2c_ref, out_ref):
    d2 = w2t_ref.shape[0]
    d = int(round(d2 ** 0.5))
    bk = ea_t_ref.shape[1]
    et = jnp.dot(w1t_ref[...], ea_t_ref[...],
                 preferred_element_type=jnp.float32)
    et = et + b1c_ref[...]
    et = jnp.where(et > 0, et, 0.01 * et)
    pt = jnp.dot(w2t_ref[...], et,
                 preferred_element_type=jnp.float32) + b2c_ref[...]
    hsbt = hs_ref[...].T
    sub8 = bk // 8
    msg_k = []
    for k in range(8):
        acc = None
        for dd in range(d):
            hrow = jnp.broadcast_to(hsbt[d * k + dd:d * k + dd + 1, :],
                                    (d, sub8))
            term = hrow * pt[dd * d:(dd + 1) * d, k * sub8:(k + 1) * sub8]
            acc = term if acc is None else acc + term
        msg_k.append(acc)
    msgt = jnp.concatenate(msg_k, axis=1)
    sub4 = bk // 4
    rr = lax.broadcasted_iota(jnp.int32, (d, 2 * d), 0)
    cc = lax.broadcasted_iota(jnp.int32, (d, 2 * d), 1)
    sel = (rr == cc).astype(jnp.float32)
    crow = (lax.broadcasted_iota(jnp.int32, (1, 2 * d), 1) == d
            ).astype(jnp.float32)
    pieces = [
        lax.dot_general(msgt[:, j * sub4:(j + 1) * sub4], sel,
                        (((0,), (0,)), ((), ())),
                        preferred_element_type=jnp.float32) + crow
        for j in range(4)
    ]
    out_ref[...] = jnp.concatenate(pieces, axis=1)


def _final_body(r_ref, *p_refs):
    out_ref = p_refs[-1]
    s = p_refs[0][...]
    for pr in p_refs[1:-1]:
        s = s + pr[...]
    d = r_ref.shape[1]
    aggr = s[:, :d] / jnp.maximum(s[:, d:d + 1], 1.0)
    out_ref[...] = r_ref[...] + aggr


def _make_gather(n_rows_per_w, n_chunks, chr_, row_off, ep_h, d, n_nodes,
                 fill_tiles):
    mesh = plsc.VectorSubcoreMesh(core_axis_name="c", subcore_axis_name="s")
    fpr = n_nodes // fill_tiles

    @functools.partial(
        pl.kernel, mesh=mesh,
        compiler_params=pltpu.CompilerParams(
            use_tc_tiling_on_sc=False, needs_layout_passes=False),
        out_type=jax.ShapeDtypeStruct((ep_h, d), jnp.float32),
        scratch_types=[
            pltpu.VMEM((chr_, _IDXW), jnp.int32),
            pltpu.VMEM((chr_ * _IDXW, d), jnp.float32),
            pltpu.VMEM((_BK,), jnp.int32),
            pltpu.VMEM_SHARED((n_nodes, d), jnp.float32),
            pltpu.SemaphoreType.DMA,
        ],
    )
    def gather_k(h_hbm, src_hbm, out_hbm, idx_v, rows_v, stage_v, h_sh, sem):
        sid = lax.axis_index("s")
        wid = sid * _NC + lax.axis_index("c")

        @pl.when(sid < fill_tiles)
        def _():
            pltpu.sync_copy(h_hbm.at[pl.ds(sid * fpr, fpr)],
                            h_sh.at[pl.ds(sid * fpr, fpr)])

        plsc.subcore_barrier()

        iota = lax.iota(jnp.int32, 16)
        evec = (iota & 7) * (_BK // 8) + (iota >> 3)
        rpb = _BK // _IDXW

        def chunk(c, carry):
            lrow = wid * n_rows_per_w + c * chr_
            grow = row_off + lrow
            g = grow // rpb
            jb = grow - g * rpb
            pltpu.sync_copy(src_hbm.at[0, pl.ds(g * _BK, _BK)], stage_v)
            for v in range(chr_ * 8):
                p0 = jb * _IDXW + v * 16
                vals = plsc.load_gather(stage_v, [evec + (p0 >> 3)])
                idx_v[v // 8, pl.ds((v % 8) * 16, 16)] = vals
            copies = [
                pltpu.async_copy(h_sh.at[idx_v.at[j]],
                                 rows_v.at[pl.ds(j * _IDXW, _IDXW)], sem)
                for j in range(chr_)
            ]
            for cp in copies:
                cp.wait()
            pltpu.sync_copy(rows_v,
                            out_hbm.at[pl.ds(lrow * _IDXW, chr_ * _IDXW)])
            return carry

        lax.fori_loop(0, n_chunks, chunk, 0)

    return gather_k


def _make_scatter(n_rows_per_w, n_chunks, chr_, row_off, np_pad, w):
    mesh = plsc.VectorSubcoreMesh(core_axis_name="c", subcore_axis_name="s")
    zr = np_pad // _NS

    @functools.partial(
        pl.kernel, mesh=mesh,
        compiler_params=pltpu.CompilerParams(
            use_tc_tiling_on_sc=False, needs_layout_passes=False),
        out_type=jax.ShapeDtypeStruct((_NC, np_pad, w), jnp.float32),
        scratch_types=[
            pltpu.VMEM((chr_, _IDXW), jnp.int32),
            pltpu.VMEM((chr_ * _IDXW, w), jnp.float32),
            pltpu.VMEM((_BK,), jnp.int32),
            pltpu.VMEM_SHARED((np_pad, w), jnp.float32),
            pltpu.SemaphoreType.DMA,
        ],
    )
    def scatter_k(msg_hbm, dst_hbm, zero_hbm, out_hbm, idx_v, buf_v, stage_v,
                  acc_sh, sem):
        cid = lax.axis_index("c")
        sid = lax.axis_index("s")
        wid = sid * _NC + cid
        pltpu.sync_copy(zero_hbm.at[pl.ds(sid * zr, zr)],
                        acc_sh.at[pl.ds(sid * zr, zr)])
        plsc.subcore_barrier()

        iota = lax.iota(jnp.int32, 16)
        evec = (iota & 3) * (_BK // 4) + (iota >> 2)
        rpb = _BK // _IDXW

        def chunk(c, carry):
            lrow = wid * n_rows_per_w + c * chr_
            grow = row_off + lrow
            g = grow // rpb
            jb = grow - g * rpb
            pltpu.sync_copy(dst_hbm.at[1, pl.ds(g * _BK, _BK)], stage_v)
            for v in range(chr_ * 8):
                p0 = jb * _IDXW + v * 16
                vals = plsc.load_gather(stage_v, [evec + (p0 >> 2)])
                idx_v[v // 8, pl.ds((v % 8) * 16, 16)] = vals
            pltpu.sync_copy(msg_hbm.at[pl.ds(lrow * _IDXW, chr_ * _IDXW)],
                            buf_v)
            for j in range(chr_):
                pltpu.sync_copy(buf_v.at[pl.ds(j * _IDXW, _IDXW)],
                                acc_sh.at[idx_v.at[j]], add=True)
            return carry

        lax.fori_loop(0, n_chunks, chunk, 0)
        plsc.subcore_barrier()
        pltpu.sync_copy(acc_sh.at[pl.ds(sid * zr, zr)],
                        out_hbm.at[cid, pl.ds(sid * zr, zr)])

    return scatter_k


def kernel(x, edge_index, edge_attr, ln_scale, ln_bias, w1, b1, w2, b2, root,
           bias):
    n, d = x.shape
    e_cnt, de = edge_attr.shape
    hdim = w1.shape[1]
    w = 2 * d

    grain = _BK * _NW
    ep = ((e_cnt + grain - 1) // grain) * grain
    rows = ep // _IDXW
    rpw = rows // _NW
    n_chunks = rpw // _CHR
    nb = ep // _BK
    np_pad = ((n + 1 + _NS * 8 - 1) // (_NS * 8)) * (_NS * 8)

    pad = ep - e_cnt
    ei_pad = jnp.concatenate(
        [edge_index,
         jnp.stack([jnp.zeros((pad,), jnp.int32),
                    jnp.full((pad,), n, jnp.int32)])], axis=1)
    ea_t = edge_attr.T

    zeros_acc = jnp.zeros((np_pad, w), jnp.float32)

    bn = 2000
    assert n % bn == 0
    h, r = pl.pallas_call(
        _norm_body,
        grid=(n // bn,),
        in_specs=[
            pl.BlockSpec((bn, d), lambda i: (i, 0)),
            pl.BlockSpec((1, d), lambda i: (0, 0)),
            pl.BlockSpec((1, d), lambda i: (0, 0)),
            pl.BlockSpec((d, d), lambda i: (0, 0)),
            pl.BlockSpec((1, d), lambda i: (0, 0)),
        ],
        out_specs=[
            pl.BlockSpec((bn, d), lambda i: (i, 0)),
            pl.BlockSpec((bn, d), lambda i: (i, 0)),
        ],
        out_shape=[
            jax.ShapeDtypeStruct((n, d), jnp.float32),
            jax.ShapeDtypeStruct((n, d), jnp.float32),
        ],
    )(x, ln_scale.reshape(1, d), ln_bias.reshape(1, d), root,
      bias.reshape(1, d))

    fill_tiles = 10
    assert n % fill_tiles == 0 and (n // fill_tiles) % 8 == 0
    rw0 = (rows * 3 // 5) // (_NW * 8) * (_NW * 8)
    stage_rows = [rw0, rows - rw0]
    chr_h = 8
    parts = []
    row_off = 0
    for srows in stage_rows:
        rpw_h = srows // _NW
        assert rpw_h % chr_h == 0
        ep_h = srows * _IDXW
        hs_h = _make_gather(rpw_h, rpw_h // chr_h, chr_h, row_off, ep_h, d,
                            n, fill_tiles)(h, ei_pad)
        hs128 = hs_h.reshape(ep_h * d // 128, 128)
        e_lo = row_off * _IDXW
        live = min(ep_h, max(0, e_cnt - e_lo))
        nb_live = (live + _BK - 1) // _BK
        msg128 = pl.pallas_call(
            _edge_body,
            grid=(nb_live,),
            in_specs=[
                pl.BlockSpec((de, _BK),
                             lambda i, o=e_lo // _BK: (0, i + o)),
                pl.BlockSpec((_BK * d // 128, 128), lambda i: (i, 0)),
                pl.BlockSpec((hdim, de), lambda i: (0, 0)),
                pl.BlockSpec((hdim, 1), lambda i: (0, 0)),
                pl.BlockSpec((d * d, hdim), lambda i: (0, 0)),
                pl.BlockSpec((d * d, 1), lambda i: (0, 0)),
            ],
            out_specs=pl.BlockSpec((_BK * w // 128, 128), lambda i: (i, 0)),
            out_shape=jax.ShapeDtypeStruct((ep_h * w // 128, 128),
                                           jnp.float32),
        )(ea_t, hs128, w1.T, b1.reshape(hdim, 1), w2.T, b2.reshape(d * d, 1))
        msg = msg128.reshape(ep_h, w)
        parts.append(
            _make_scatter(rpw_h, rpw_h // chr_h, chr_h, row_off, np_pad,
                          w)(msg, ei_pad, zeros_acc))
        row_off += srows
    nh = len(parts)

    out = pl.pallas_call(
        _final_body,
        grid=(n // bn,),
        in_specs=[pl.BlockSpec((bn, d), lambda i: (i, 0))] + [
            pl.BlockSpec((bn, w), lambda i: (i, 0))
            for _ in range(2 * nh)
        ],
        out_specs=pl.BlockSpec((bn, d), lambda i: (i, 0)),
        out_shape=jax.ShapeDtypeStruct((n, d), jnp.float32),
    )(r, *[p[c, :n] for p in parts for c in range(2)])
    return out
